# Initial kernel scaffold; baseline (speedup 1.0000x reference)
import jax
import jax.numpy as jnp
from jax.experimental import pallas as pl


def kernel(tokens, table):
    raise NotImplementedError("write your pallas kernel here")


# speedup vs baseline: 1.6844x; 1.6844x over previous
import functools

import jax
import jax.numpy as jnp
from jax import lax
from jax.experimental import pallas as pl
from jax.experimental.pallas import tpu as pltpu
from jax.experimental.pallas import tpu_sc as plsc

NC = 2
NS = 16
NW = NC * NS

CHUNK = 128


def _make_gather(vocab, d, n_chunks):
  mesh = plsc.VectorSubcoreMesh(core_axis_name="c", subcore_axis_name="s")

  @functools.partial(
      pl.kernel,
      out_type=jax.ShapeDtypeStruct((NW, n_chunks, CHUNK, d), jnp.float32),
      mesh=mesh,
      compiler_params=pltpu.CompilerParams(use_tc_tiling_on_sc=False),
      scratch_types=[
          pltpu.VMEM((n_chunks, CHUNK), jnp.int32),
          pltpu.VMEM((CHUNK, d), jnp.float32),
          pltpu.SemaphoreType.DMA,
      ],
  )
  def gather(table_hbm, idx_hbm, out_hbm, idx_v, buf, gsem):
    wid = lax.axis_index("s") * NC + lax.axis_index("c")
    pltpu.sync_copy(idx_hbm.at[wid], idx_v)

    def step(j, carry):
      pltpu.async_copy(table_hbm.at[idx_v.at[j]], buf, gsem).wait()
      pltpu.sync_copy(buf, out_hbm.at[wid, j])
      return carry

    lax.fori_loop(0, n_chunks, step, 0)

  return gather


def kernel(tokens, table):
  b, s = tokens.shape
  vocab, d = table.shape
  total = b * s
  assert total % (NW * CHUNK) == 0
  n_chunks = total // (NW * CHUNK)
  idx = tokens.reshape(NW, n_chunks, CHUNK).astype(jnp.int32)
  out = _make_gather(vocab, d, n_chunks)(table, idx)
  return out.reshape(b, s, d)


# speedup vs baseline: 1.8745x; 1.1129x over previous
import functools

import jax
import jax.numpy as jnp
from jax import lax
from jax.experimental import pallas as pl
from jax.experimental.pallas import tpu as pltpu
from jax.experimental.pallas import tpu_sc as plsc

NC = 2
NS = 16
NW = NC * NS

CHUNK = 128
NBUF = 8
LEAD = 4


def _make_gather(vocab, d, n_chunks):
  mesh = plsc.VectorSubcoreMesh(core_axis_name="c", subcore_axis_name="s")
  assert n_chunks % NBUF == 0
  n_outer = n_chunks // NBUF

  @functools.partial(
      pl.kernel,
      out_type=jax.ShapeDtypeStruct((NW, n_chunks, CHUNK, d), jnp.float32),
      mesh=mesh,
      compiler_params=pltpu.CompilerParams(use_tc_tiling_on_sc=False),
      scratch_types=[
          pltpu.VMEM((n_chunks, CHUNK), jnp.int32),
          pltpu.VMEM((NBUF, CHUNK, d), jnp.float32),
          [pltpu.SemaphoreType.DMA] * NBUF,
          [pltpu.SemaphoreType.DMA] * NBUF,
      ],
  )
  def gather(table_hbm, idx_hbm, out_hbm, idx_v, bufs, gsems, wsems):
    wid = lax.axis_index("s") * NC + lax.axis_index("c")
    pltpu.sync_copy(idx_hbm.at[wid], idx_v)

    def start_gather(j, b):
      pltpu.async_copy(table_hbm.at[idx_v.at[j]], bufs.at[b], gsems[b])

    def wait_gather(b):
      pltpu.make_async_copy(table_hbm.at[idx_v.at[0]], bufs.at[b],
                            gsems[b]).wait()

    def wait_writeback(b):
      pltpu.make_async_copy(bufs.at[b], out_hbm.at[wid, 0], wsems[b]).wait()

    for b in range(LEAD):
      start_gather(b, b)

    def outer(o, carry):
      j0 = o * NBUF
      for i in range(NBUF):
        b = i % NBUF
        bp = (i + LEAD) % NBUF
        j = j0 + i
        if i + LEAD >= NBUF:
          wait_writeback(bp)
        else:
          @pl.when(o >= 1)
          def _():
            wait_writeback(bp)

        @pl.when(j + LEAD < n_chunks)
        def _():
          start_gather(j + LEAD, bp)

        wait_gather(b)
        pltpu.async_copy(bufs.at[b], out_hbm.at[wid, j], wsems[b])
      return carry

    lax.fori_loop(0, n_outer, outer, 0)

    for k in range(NBUF - LEAD):
      j = n_chunks - (NBUF - LEAD) + k
      wait_writeback(j % NBUF)

  return gather


def kernel(tokens, table):
  b, s = tokens.shape
  vocab, d = table.shape
  total = b * s
  assert total % (NW * CHUNK) == 0
  n_chunks = total // (NW * CHUNK)
  idx = tokens.reshape(NW, n_chunks, CHUNK).astype(jnp.int32)
  out = _make_gather(vocab, d, n_chunks)(table, idx)
  return out.reshape(b, s, d)


# speedup vs baseline: 1.8803x; 1.0031x over previous
import functools

import jax
import jax.numpy as jnp
from jax import lax
from jax.experimental import pallas as pl
from jax.experimental.pallas import tpu as pltpu
from jax.experimental.pallas import tpu_sc as plsc

NC = 2
NS = 16
NW = NC * NS

CHUNK = 128
NBUF = 8
LEAD = 4


def _make_gather(b_total, s_total, d, n_chunks):
  mesh = plsc.VectorSubcoreMesh(core_axis_name="c", subcore_axis_name="s")
  assert n_chunks % NBUF == 0
  n_outer = n_chunks // NBUF
  bt_per_s = b_total // CHUNK
  assert bt_per_s & (bt_per_s - 1) == 0
  bt_bits = bt_per_s.bit_length() - 1

  @functools.partial(
      pl.kernel,
      out_type=jax.ShapeDtypeStruct((b_total, s_total, d), jnp.float32),
      mesh=mesh,
      compiler_params=pltpu.CompilerParams(use_tc_tiling_on_sc=False),
      scratch_types=[
          pltpu.VMEM((n_chunks, CHUNK), jnp.int32),
          pltpu.VMEM((NBUF, CHUNK, d), jnp.float32),
          [pltpu.SemaphoreType.DMA] * NBUF,
          [pltpu.SemaphoreType.DMA] * NBUF,
      ],
  )
  def gather(table_hbm, idx_hbm, out_hbm, idx_v, bufs, gsems, wsems):
    wid = lax.axis_index("s") * NC + lax.axis_index("c")
    pltpu.sync_copy(idx_hbm.at[wid], idx_v)
    j0w = wid * n_chunks

    def out_slice(c):
      j = j0w + c
      s = j >> bt_bits
      b0 = (j & (bt_per_s - 1)) * CHUNK
      return out_hbm.at[pl.ds(b0, CHUNK), s, :]

    def start_gather(c, b):
      pltpu.async_copy(table_hbm.at[idx_v.at[c]], bufs.at[b], gsems[b])

    def wait_gather(b):
      pltpu.make_async_copy(table_hbm.at[idx_v.at[0]], bufs.at[b],
                            gsems[b]).wait()

    def start_writeback(c, b):
      pltpu.async_copy(bufs.at[b], out_slice(c), wsems[b])

    def wait_writeback(b):
      pltpu.make_async_copy(bufs.at[b], out_slice(0), wsems[b]).wait()

    for b in range(LEAD):
      start_gather(b, b)

    def outer(o, carry):
      c0 = o * NBUF
      for i in range(NBUF):
        b = i % NBUF
        bp = (i + LEAD) % NBUF
        c = c0 + i
        if i + LEAD >= NBUF:
          wait_writeback(bp)
        else:
          @pl.when(o >= 1)
          def _():
            wait_writeback(bp)

        @pl.when(c + LEAD < n_chunks)
        def _():
          start_gather(c + LEAD, bp)

        wait_gather(b)
        start_writeback(c, b)
      return carry

    lax.fori_loop(0, n_outer, outer, 0)

    for k in range(NBUF - LEAD):
      wait_writeback((n_chunks - (NBUF - LEAD) + k) % NBUF)

  return gather


def kernel(tokens, table):
  b, s = tokens.shape
  vocab, d = table.shape
  total = b * s
  assert total % (NW * CHUNK) == 0
  n_chunks = total // (NW * CHUNK)
  idx = tokens.T.reshape(NW, n_chunks, CHUNK).astype(jnp.int32)
  return _make_gather(b, s, d, n_chunks)(table, idx)
